# baseline (speedup 1.0000x reference)
import functools
import math

import jax
import jax.numpy as jnp
from jax import lax
from jax.experimental import pallas as pl
from jax.experimental.pallas import tpu as pltpu
from jax.experimental.pallas import tpu_sc as plsc

D_MODEL = 64
SCALE = math.sqrt(D_MODEL)
NUM_CORES = 2
NUM_SUBCORES = 16
NUM_WORKERS = NUM_CORES * NUM_SUBCORES
CHUNK = 512
GATHER = 128
NG = CHUNK // GATHER


@functools.partial(jax.jit, static_argnums=(2,))
def _embedding_lookup(idx, table, batch):
    b_per_w = batch // NUM_WORKERS
    n_chunks = b_per_w // CHUNK
    mesh = plsc.VectorSubcoreMesh(core_axis_name="c", subcore_axis_name="s")

    @functools.partial(
        pl.kernel,
        mesh=mesh,
        out_type=jax.ShapeDtypeStruct((batch, D_MODEL), jnp.float32),
        scratch_types=[
            pltpu.VMEM((NG, GATHER), jnp.int32),
            pltpu.VMEM((CHUNK, D_MODEL), jnp.float32),
            pltpu.SemaphoreType.DMA,
        ],
        compiler_params=pltpu.CompilerParams(use_tc_tiling_on_sc=False),
    )
    def k(idx_hbm, table_hbm, out_hbm, idx_v, rows_v, sem):
        wid = lax.axis_index("s") * NUM_CORES + lax.axis_index("c")
        base_g = wid * (b_per_w // GATHER)

        def chunk_body(c, carry):
            g0 = base_g + c * NG
            pltpu.sync_copy(idx_hbm.at[pl.ds(g0, NG)], idx_v)
            copies = [
                pltpu.async_copy(
                    table_hbm.at[idx_v.at[j]],
                    rows_v.at[pl.ds(j * GATHER, GATHER)],
                    sem,
                )
                for j in range(NG)
            ]
            for cp in copies:
                cp.wait()

            def scale_body(i, carry2):
                for j in range(D_MODEL // 16):
                    sl = pl.ds(j * 16, 16)
                    rows_v[i, sl] = rows_v[i, sl] * SCALE
                return carry2

            lax.fori_loop(0, CHUNK, scale_body, 0)
            pltpu.sync_copy(rows_v, out_hbm.at[pl.ds(g0 * GATHER, CHUNK)])
            return carry

        lax.fori_loop(0, n_chunks, chunk_body, 0)

    return k(idx, table)


def kernel(x, table):
    b0, b1 = x.shape
    batch = b0 * b1
    idx = x.astype(jnp.int32).reshape(batch // GATHER, GATHER)
    out = _embedding_lookup(idx, table, batch)
    return out.reshape(b0, b1, D_MODEL)


# speedup vs baseline: 1.1404x; 1.1404x over previous
import functools
import math

import jax
import jax.numpy as jnp
from jax import lax
from jax.experimental import pallas as pl
from jax.experimental.pallas import tpu as pltpu
from jax.experimental.pallas import tpu_sc as plsc

D_MODEL = 64
SCALE = math.sqrt(D_MODEL)
NUM_CORES = 2
NUM_SUBCORES = 16
NUM_WORKERS = NUM_CORES * NUM_SUBCORES
CHUNK = 512
GATHER = 128
NG = CHUNK // GATHER
NBUF = 2
CHUNK_BYTES = CHUNK * D_MODEL * 4


@functools.partial(jax.jit, static_argnums=(2,))
def _embedding_lookup(idx, table, batch):
    b_per_w = batch // NUM_WORKERS
    n_chunks = b_per_w // CHUNK
    assert n_chunks % NBUF == 0
    idx_rows = b_per_w // GATHER
    mesh = plsc.VectorSubcoreMesh(core_axis_name="c", subcore_axis_name="s")

    @functools.partial(
        pl.kernel,
        mesh=mesh,
        out_type=jax.ShapeDtypeStruct((batch, D_MODEL), jnp.float32),
        scratch_types=[
            pltpu.VMEM((idx_rows, GATHER), jnp.int32),
            [pltpu.VMEM((CHUNK, D_MODEL), jnp.float32) for _ in range(NBUF)],
            [pltpu.SemaphoreType.DMA for _ in range(NBUF)],
            [pltpu.SemaphoreType.DMA for _ in range(NBUF)],
        ],
        compiler_params=pltpu.CompilerParams(use_tc_tiling_on_sc=False),
    )
    def k(idx_hbm, table_hbm, out_hbm, idx_v, rows, gsems, osems):
        wid = lax.axis_index("s") * NUM_CORES + lax.axis_index("c")
        base_g = wid * idx_rows

        pltpu.sync_copy(idx_hbm.at[pl.ds(base_g, idx_rows)], idx_v)

        def fire_gather(cc, b):
            for j in range(NG):
                pltpu.async_copy(
                    table_hbm.at[idx_v.at[cc * NG + j]],
                    rows[b].at[pl.ds(j * GATHER, GATHER)],
                    gsems[b],
                )

        def wait_chunk(sem, b):
            pltpu.make_async_copy(table_hbm.at[pl.ds(0, CHUNK)], rows[b], sem).wait()

        def scale(b):
            def body(i, carry):
                for r in range(4):
                    for j in range(D_MODEL // 16):
                        sl = pl.ds(j * 16, 16)
                        rows[b][i * 4 + r, sl] = rows[b][i * 4 + r, sl] * SCALE
                return carry

            lax.fori_loop(0, CHUNK // 4, body, 0)

        for b in range(NBUF):
            fire_gather(b, b)

        def loop_body(c, carry):
            for b in range(NBUF):
                cc = c + b
                wait_chunk(gsems[b], b)
                scale(b)
                pltpu.async_copy(
                    rows[b],
                    out_hbm.at[pl.ds((base_g + cc * NG) * GATHER, CHUNK)],
                    osems[b],
                )
                wait_chunk(osems[b], b)

                @pl.when(cc + NBUF < n_chunks)
                def _():
                    fire_gather(cc + NBUF, b)

            return carry

        lax.fori_loop(0, n_chunks // NBUF, lambda i, c: loop_body(i * NBUF, c), 0)

    return k(idx, table)


def kernel(x, table):
    b0, b1 = x.shape
    batch = b0 * b1
    idx = x.astype(jnp.int32).reshape(batch // GATHER, GATHER)
    out = _embedding_lookup(idx, table, batch)
    return out.reshape(b0, b1, D_MODEL)
